# Initial kernel scaffold; baseline (speedup 1.0000x reference)
import jax
import jax.numpy as jnp
from jax.experimental import pallas as pl


def kernel(x, edge_index, edge_weight, W1, b1, W2, b2):
    raise NotImplementedError("write your pallas kernel here")


# speedup vs baseline: 4.7984x; 4.7984x over previous
import functools

import jax
import jax.numpy as jnp
from jax import lax
from jax.experimental import pallas as pl
from jax.experimental.pallas import tpu as pltpu
from jax.experimental.pallas import tpu_sc as plsc

N_NODES = 10000
N_EDGES = 320000
F = 128
N_CLASSES = 40

NC, NS, NW = 2, 16, 32
NP = 10240
NPR = NP // F
NH = NP // NC
ACC = 5376
ACCSEG = ACC // NS
SEG = NH // NS
SEGR = NP // NS // F
K = 128
NCHUNK = 80
EP = NW * NCHUNK * K

_f32 = jnp.float32
_i32 = jnp.int32


def _rsqrt16(x):
    i = plsc.bitcast(x, _i32)
    i = jnp.int32(0x5F3759DF) - lax.shift_right_logical(i, 1)
    y = plsc.bitcast(i, _f32)
    for _ in range(3):
        y = y * (1.5 - 0.5 * x * y * y)
    return y


def _zero_rows(ref, nrows):
    z = jnp.zeros((16,), _f32)

    @pl.loop(0, nrows)
    def _(r):
        for j in range(F // 16):
            ref[r, pl.ds(16 * j, 16)] = z


def _edge_block(c, h_h, srcb, dstb, ewb, dinv_v, wsrc_v, gb, accsh,
                gsem, ssem):
    base = c * NH
    lanes = lax.iota(_i32, 16)

    @pl.loop(0, NCHUNK)
    def _(ch):
        for i in range(K // 16):
            sl = pl.ds(16 * i, 16)
            v = srcb[ch, sl]
            sv = lax.bitwise_and(v, 0xFFFF)
            dl = lax.shift_right_logical(v, 16) - base
            ok = jnp.logical_and(dl >= 0, dl < NH)
            dstb[ch, sl] = jnp.where(ok, dl, NH + 16 * i + lanes)
            srcb[ch, sl] = sv
            rv = lax.shift_right_logical(sv, 7)
            cv = lax.bitwise_and(sv, 127)
            dv = plsc.load_gather(dinv_v, [rv, cv])
            wsrc_v[sl] = ewb[ch, sl] * dv

        pltpu.async_copy(h_h.at[srcb.at[ch]], gb, gsem).wait()

        @pl.loop(0, K)
        def _(e):
            wv = plsc.load_gather(wsrc_v, [jnp.full((16,), e, _i32)])
            for j in range(F // 16):
                gb[e, pl.ds(16 * j, 16)] = gb[e, pl.ds(16 * j, 16)] * wv

        pltpu.async_copy(gb, accsh.at[dstb.at[ch]], ssem, add=True).wait()


def _edge_phase(s, c, sd3_h, ew3_h, h_h, parts_h,
                srcb, dstb, ewb, dinv_v, wsrc_v, gb, accsh, gsem, ssem):
    _zero_rows(gb, K)
    pltpu.sync_copy(gb, accsh.at[pl.ds(s * ACCSEG, K)])
    pltpu.sync_copy(gb, accsh.at[pl.ds(s * ACCSEG + K, K)])
    pltpu.sync_copy(gb.at[pl.ds(0, ACCSEG - 2 * K)],
                    accsh.at[pl.ds(s * ACCSEG + 2 * K, ACCSEG - 2 * K)])
    plsc.subcore_barrier()

    for m in range(2):
        blk = s * 2 + m
        pltpu.sync_copy(sd3_h.at[blk], srcb)
        pltpu.sync_copy(ew3_h.at[blk], ewb)
        _edge_block(c, h_h, srcb, dstb, ewb, dinv_v, wsrc_v, gb, accsh,
                    gsem, ssem)

    plsc.subcore_barrier()
    pltpu.sync_copy(accsh.at[pl.ds(s * SEG, SEG)],
                    parts_h.at[c, pl.ds(s * SEG, SEG)])


_SC_PARAMS = pltpu.CompilerParams(needs_layout_passes=False)
_MESH = plsc.VectorSubcoreMesh(core_axis_name="c", subcore_axis_name="s")

_EDGE_SCRATCH = [
    pltpu.VMEM((NCHUNK, K), _i32),
    pltpu.VMEM((NCHUNK, K), _i32),
    pltpu.VMEM((NCHUNK, K), _f32),
    pltpu.VMEM((NPR, F), _f32),
    pltpu.VMEM((K,), _f32),
    pltpu.VMEM((K, F), _f32),
    pltpu.VMEM_SHARED((ACC, F), _f32),
    pltpu.SemaphoreType.DMA,
    pltpu.SemaphoreType.DMA,
]


@functools.partial(
    pl.kernel,
    out_type=[
        jax.ShapeDtypeStruct((NC, NH, F), _f32),
        jax.ShapeDtypeStruct((NPR, F), _f32),
    ],
    mesh=_MESH,
    compiler_params=_SC_PARAMS,
    scratch_types=_EDGE_SCRATCH + [
        pltpu.VMEM((NPR, F), _f32),
        pltpu.VMEM((SEGR, F), _f32),
        pltpu.VMEM((SEGR, F), _f32),
        pltpu.VMEM_SHARED((NPR, F), _f32),
    ],
)
def _sc_layer1(sd3_h, ew3_h, h_h, parts_h, dinv_h,
               srcb, dstb, ewb, dinv_v, wsrc_v, gb, accsh, gsem, ssem,
               deg_v, red_v, tmp_v, dinvsh):
    c = lax.axis_index("c")
    s = lax.axis_index("s")

    _zero_rows(deg_v, NPR)
    for m in range(2):
        blk = s * 2 + m
        pltpu.sync_copy(sd3_h.at[blk], dstb)
        pltpu.sync_copy(ew3_h.at[blk], ewb)

        @pl.loop(0, NCHUNK)
        def _(ch):
            for i in range(K // 16):
                sl = pl.ds(16 * i, 16)
                iv = lax.shift_right_logical(dstb[ch, sl], 16)
                wv = ewb[ch, sl]
                rv = lax.shift_right_logical(iv, 7)
                cv = lax.bitwise_and(iv, 127)
                plsc.addupdate_scatter(deg_v, [rv, cv], wv)

    pltpu.sync_copy(deg_v, accsh.at[pl.ds(s * NPR, NPR)])
    plsc.subcore_barrier()

    _zero_rows(red_v, SEGR)
    for r in range(NS):
        pltpu.sync_copy(accsh.at[pl.ds(r * NPR + s * SEGR, SEGR)], tmp_v)

        @pl.loop(0, SEGR)
        def _(i):
            for j in range(F // 16):
                red_v[i, pl.ds(16 * j, 16)] = (red_v[i, pl.ds(16 * j, 16)]
                                               + tmp_v[i, pl.ds(16 * j, 16)])

    @pl.loop(0, SEGR)
    def _(i):
        for j in range(F // 16):
            tmp_v[i, pl.ds(16 * j, 16)] = _rsqrt16(
                red_v[i, pl.ds(16 * j, 16)] + 1.0)

    pltpu.sync_copy(tmp_v, dinvsh.at[pl.ds(s * SEGR, SEGR)])
    plsc.subcore_barrier()

    @pl.when(jnp.logical_and(c == 0, s == 0))
    def _():
        pltpu.sync_copy(dinvsh, dinv_h)

    pltpu.sync_copy(dinvsh, dinv_v)

    _edge_phase(s, c, sd3_h, ew3_h, h_h, parts_h,
                srcb, dstb, ewb, dinv_v, wsrc_v, gb, accsh, gsem, ssem)


@functools.partial(
    pl.kernel,
    out_type=jax.ShapeDtypeStruct((NC, NH, F), _f32),
    mesh=_MESH,
    compiler_params=_SC_PARAMS,
    scratch_types=_EDGE_SCRATCH,
)
def _sc_layer2(sd3_h, ew3_h, h_h, dinv_hin, parts_h,
               srcb, dstb, ewb, dinv_v, wsrc_v, gb, accsh, gsem, ssem):
    c = lax.axis_index("c")
    s = lax.axis_index("s")
    pltpu.sync_copy(dinv_hin, dinv_v)
    _edge_phase(s, c, sd3_h, ew3_h, h_h, parts_h,
                srcb, dstb, ewb, dinv_v, wsrc_v, gb, accsh, gsem, ssem)


_BR = 1024


def _mm_body(x_ref, w_ref, o_ref):
    o_ref[...] = jnp.dot(x_ref[...], w_ref[...], preferred_element_type=_f32)


_tc_matmul = pl.pallas_call(
    _mm_body,
    grid=(NP // _BR,),
    in_specs=[
        pl.BlockSpec((_BR, F), lambda i: (i, 0)),
        pl.BlockSpec((F, F), lambda i: (0, 0)),
    ],
    out_specs=pl.BlockSpec((_BR, F), lambda i: (i, 0)),
    out_shape=jax.ShapeDtypeStruct((NP, F), _f32),
)


def _comb1_body(agg_ref, h_ref, di_ref, b_ref, o_ref):
    di = di_ref[...]
    o_ref[...] = jnp.maximum(
        agg_ref[...] * di + h_ref[...] * (di * di) + b_ref[...], 0.0)


_tc_comb1 = pl.pallas_call(
    _comb1_body,
    grid=(NP // _BR,),
    in_specs=[
        pl.BlockSpec((_BR, F), lambda i: (i, 0)),
        pl.BlockSpec((_BR, F), lambda i: (i, 0)),
        pl.BlockSpec((_BR, 1), lambda i: (i, 0)),
        pl.BlockSpec((1, F), lambda i: (0, 0)),
    ],
    out_specs=pl.BlockSpec((_BR, F), lambda i: (i, 0)),
    out_shape=jax.ShapeDtypeStruct((NP, F), _f32),
)


def _comb2_body(agg_ref, h_ref, di_ref, w2_ref, b2_ref, o_ref):
    di = di_ref[...]
    a = agg_ref[...] * di + h_ref[...] * (di * di)
    o_ref[...] = (jnp.dot(a, w2_ref[...], preferred_element_type=_f32)
                  + b2_ref[...])


_tc_comb2 = pl.pallas_call(
    _comb2_body,
    grid=(NP // _BR,),
    in_specs=[
        pl.BlockSpec((_BR, F), lambda i: (i, 0)),
        pl.BlockSpec((_BR, F), lambda i: (i, 0)),
        pl.BlockSpec((_BR, 1), lambda i: (i, 0)),
        pl.BlockSpec((F, N_CLASSES), lambda i: (0, 0)),
        pl.BlockSpec((1, N_CLASSES), lambda i: (0, 0)),
    ],
    out_specs=pl.BlockSpec((_BR, N_CLASSES), lambda i: (i, 0)),
    out_shape=jax.ShapeDtypeStruct((NP, N_CLASSES), _f32),
)


def kernel(x, edge_index, edge_weight, W1, b1, W2, b2):
    src = edge_index[0].astype(_i32)
    dst = edge_index[1].astype(_i32)
    pad = EP - N_EDGES
    sd = jnp.bitwise_or(jnp.left_shift(dst, 16), src)
    sd3 = jnp.concatenate([sd, jnp.zeros((pad,), _i32)]).reshape(NW, NCHUNK, K)
    ew3 = jnp.concatenate([edge_weight.astype(_f32),
                           jnp.zeros((pad,), _f32)]).reshape(NW, NCHUNK, K)
    x_p = jnp.zeros((NP, F), _f32).at[:N_NODES].set(x)

    h1 = _tc_matmul(x_p, W1)
    parts1, dinv2d = _sc_layer1(sd3, ew3, h1)
    agg1 = parts1.reshape(NP, F)
    dinv_col = dinv2d.reshape(NP, 1)
    h1r = _tc_comb1(agg1, h1, dinv_col, b1.reshape(1, F))
    parts2 = _sc_layer2(sd3, ew3, h1r, dinv2d)
    agg2 = parts2.reshape(NP, F)
    out_p = _tc_comb2(agg2, h1r, dinv_col, W2, b2.reshape(1, N_CLASSES))
    return out_p[:N_NODES]


# speedup vs baseline: 5.4658x; 1.1391x over previous
import functools

import jax
import jax.numpy as jnp
from jax import lax
from jax.experimental import pallas as pl
from jax.experimental.pallas import tpu as pltpu
from jax.experimental.pallas import tpu_sc as plsc

N_NODES = 10000
N_EDGES = 320000
F = 128
N_CLASSES = 40

NC, NS, NW = 2, 16, 32
NP = 10240
NPR = NP // F
NH = NP // NC
ACC = 5376
ACCSEG = ACC // NS
SEG = NH // NS
SEGR = NP // NS // F
K = 128
NCHUNK = 80
EP = NW * NCHUNK * K

_f32 = jnp.float32
_i32 = jnp.int32


def _rsqrt16(x):
    i = plsc.bitcast(x, _i32)
    i = jnp.int32(0x5F3759DF) - lax.shift_right_logical(i, 1)
    y = plsc.bitcast(i, _f32)
    for _ in range(3):
        y = y * (1.5 - 0.5 * x * y * y)
    return y


def _zero_rows(ref, nrows):
    z = jnp.zeros((16,), _f32)

    @pl.loop(0, nrows)
    def _(r):
        for j in range(F // 16):
            ref[r, pl.ds(16 * j, 16)] = z


def _edge_block(c, h_h, srcb, dstb, ewb, dinv_v, gb0, gb1, accsh,
                gsem0, gsem1, ssem):
    base = c * NH
    lanes = lax.iota(_i32, 16)

    @pl.loop(0, NCHUNK)
    def _(ch):
        for i in range(K // 16):
            sl = pl.ds(16 * i, 16)
            v = srcb[ch, sl]
            sv = lax.bitwise_and(v, 0xFFFF)
            dl = lax.shift_right_logical(v, 16) - base
            ok = jnp.logical_and(dl >= 0, dl < NH)
            dstb[ch, sl] = jnp.where(ok, dl, NH + 16 * i + lanes)
            srcb[ch, sl] = sv
            rv = lax.shift_right_logical(sv, 7)
            cv = lax.bitwise_and(sv, 127)
            dv = plsc.load_gather(dinv_v, [rv, cv])
            ewb[ch, sl] = ewb[ch, sl] * dv

    def _scale_scatter(ch, buf):
        @pl.loop(0, K)
        def _(e):
            wv = plsc.load_gather(ewb, [jnp.full((16,), ch, _i32),
                                        jnp.full((16,), e, _i32)])
            for j in range(F // 16):
                buf[e, pl.ds(16 * j, 16)] = buf[e, pl.ds(16 * j, 16)] * wv

        pltpu.async_copy(buf, accsh.at[dstb.at[ch]], ssem, add=True).wait()

    pltpu.async_copy(h_h.at[srcb.at[0]], gb0, gsem0)

    @pl.loop(0, NCHUNK // 2)
    def _(g):
        ch0 = 2 * g
        pltpu.async_copy(h_h.at[srcb.at[ch0 + 1]], gb1, gsem1)
        pltpu.make_async_copy(h_h.at[srcb.at[ch0]], gb0, gsem0).wait()
        _scale_scatter(ch0, gb0)

        @pl.when(ch0 + 2 < NCHUNK)
        def _():
            pltpu.async_copy(h_h.at[srcb.at[ch0 + 2]], gb0, gsem0)

        pltpu.make_async_copy(h_h.at[srcb.at[ch0 + 1]], gb1, gsem1).wait()
        _scale_scatter(ch0 + 1, gb1)


def _edge_phase(s, c, sd3_h, ew3_h, h_h, parts_h,
                srcb, dstb, ewb, dinv_v, gb0, gb1, accsh, gsem0, gsem1, ssem):
    _zero_rows(gb0, K)
    pltpu.sync_copy(gb0, accsh.at[pl.ds(s * ACCSEG, K)])
    pltpu.sync_copy(gb0, accsh.at[pl.ds(s * ACCSEG + K, K)])
    pltpu.sync_copy(gb0.at[pl.ds(0, ACCSEG - 2 * K)],
                    accsh.at[pl.ds(s * ACCSEG + 2 * K, ACCSEG - 2 * K)])
    plsc.subcore_barrier()

    for m in range(2):
        blk = s * 2 + m
        pltpu.sync_copy(sd3_h.at[blk], srcb)
        pltpu.sync_copy(ew3_h.at[blk], ewb)
        _edge_block(c, h_h, srcb, dstb, ewb, dinv_v, gb0, gb1, accsh,
                    gsem0, gsem1, ssem)

    plsc.subcore_barrier()
    pltpu.sync_copy(accsh.at[pl.ds(s * SEG, SEG)],
                    parts_h.at[c, pl.ds(s * SEG, SEG)])


_SC_PARAMS = pltpu.CompilerParams(needs_layout_passes=False)
_MESH = plsc.VectorSubcoreMesh(core_axis_name="c", subcore_axis_name="s")

_EDGE_SCRATCH = [
    pltpu.VMEM((NCHUNK, K), _i32),
    pltpu.VMEM((NCHUNK, K), _i32),
    pltpu.VMEM((NCHUNK, K), _f32),
    pltpu.VMEM((NPR, F), _f32),
    pltpu.VMEM((K, F), _f32),
    pltpu.VMEM((K, F), _f32),
    pltpu.VMEM_SHARED((ACC, F), _f32),
    pltpu.SemaphoreType.DMA,
    pltpu.SemaphoreType.DMA,
    pltpu.SemaphoreType.DMA,
]


@functools.partial(
    pl.kernel,
    out_type=[
        jax.ShapeDtypeStruct((NC, NH, F), _f32),
        jax.ShapeDtypeStruct((NPR, F), _f32),
    ],
    mesh=_MESH,
    compiler_params=_SC_PARAMS,
    scratch_types=_EDGE_SCRATCH + [
        pltpu.VMEM((NPR, F), _f32),
        pltpu.VMEM((SEGR, F), _f32),
        pltpu.VMEM((SEGR, F), _f32),
        pltpu.VMEM_SHARED((NPR, F), _f32),
    ],
)
def _sc_layer1(sd3_h, ew3_h, h_h, parts_h, dinv_h,
               srcb, dstb, ewb, dinv_v, gb0, gb1, accsh, gsem0, gsem1, ssem,
               deg_v, red_v, tmp_v, dinvsh):
    c = lax.axis_index("c")
    s = lax.axis_index("s")

    _zero_rows(deg_v, NPR)
    for m in range(2):
        blk = s * 2 + m
        pltpu.sync_copy(sd3_h.at[blk], dstb)
        pltpu.sync_copy(ew3_h.at[blk], ewb)

        @pl.loop(0, NCHUNK)
        def _(ch):
            for i in range(K // 16):
                sl = pl.ds(16 * i, 16)
                iv = lax.shift_right_logical(dstb[ch, sl], 16)
                wv = ewb[ch, sl]
                rv = lax.shift_right_logical(iv, 7)
                cv = lax.bitwise_and(iv, 127)
                plsc.addupdate_scatter(deg_v, [rv, cv], wv)

    pltpu.sync_copy(deg_v, accsh.at[pl.ds(s * NPR, NPR)])
    plsc.subcore_barrier()

    _zero_rows(red_v, SEGR)
    for r in range(NS):
        pltpu.sync_copy(accsh.at[pl.ds(r * NPR + s * SEGR, SEGR)], tmp_v)

        @pl.loop(0, SEGR)
        def _(i):
            for j in range(F // 16):
                red_v[i, pl.ds(16 * j, 16)] = (red_v[i, pl.ds(16 * j, 16)]
                                               + tmp_v[i, pl.ds(16 * j, 16)])

    @pl.loop(0, SEGR)
    def _(i):
        for j in range(F // 16):
            tmp_v[i, pl.ds(16 * j, 16)] = _rsqrt16(
                red_v[i, pl.ds(16 * j, 16)] + 1.0)

    pltpu.sync_copy(tmp_v, dinvsh.at[pl.ds(s * SEGR, SEGR)])
    plsc.subcore_barrier()

    @pl.when(jnp.logical_and(c == 0, s == 0))
    def _():
        pltpu.sync_copy(dinvsh, dinv_h)

    pltpu.sync_copy(dinvsh, dinv_v)

    _edge_phase(s, c, sd3_h, ew3_h, h_h, parts_h,
                srcb, dstb, ewb, dinv_v, gb0, gb1, accsh, gsem0, gsem1, ssem)


@functools.partial(
    pl.kernel,
    out_type=jax.ShapeDtypeStruct((NC, NH, F), _f32),
    mesh=_MESH,
    compiler_params=_SC_PARAMS,
    scratch_types=_EDGE_SCRATCH,
)
def _sc_layer2(sd3_h, ew3_h, h_h, dinv_hin, parts_h,
               srcb, dstb, ewb, dinv_v, gb0, gb1, accsh, gsem0, gsem1, ssem):
    c = lax.axis_index("c")
    s = lax.axis_index("s")
    pltpu.sync_copy(dinv_hin, dinv_v)
    _edge_phase(s, c, sd3_h, ew3_h, h_h, parts_h,
                srcb, dstb, ewb, dinv_v, gb0, gb1, accsh, gsem0, gsem1, ssem)


_BR = 1024


def _mm_body(x_ref, w_ref, o_ref):
    o_ref[...] = jnp.dot(x_ref[...], w_ref[...], preferred_element_type=_f32)


_tc_matmul = pl.pallas_call(
    _mm_body,
    grid=(NP // _BR,),
    in_specs=[
        pl.BlockSpec((_BR, F), lambda i: (i, 0)),
        pl.BlockSpec((F, F), lambda i: (0, 0)),
    ],
    out_specs=pl.BlockSpec((_BR, F), lambda i: (i, 0)),
    out_shape=jax.ShapeDtypeStruct((NP, F), _f32),
)


def _comb1_body(agg_ref, h_ref, di_ref, b_ref, o_ref):
    di = di_ref[...]
    o_ref[...] = jnp.maximum(
        agg_ref[...] * di + h_ref[...] * (di * di) + b_ref[...], 0.0)


_tc_comb1 = pl.pallas_call(
    _comb1_body,
    grid=(NP // _BR,),
    in_specs=[
        pl.BlockSpec((_BR, F), lambda i: (i, 0)),
        pl.BlockSpec((_BR, F), lambda i: (i, 0)),
        pl.BlockSpec((_BR, 1), lambda i: (i, 0)),
        pl.BlockSpec((1, F), lambda i: (0, 0)),
    ],
    out_specs=pl.BlockSpec((_BR, F), lambda i: (i, 0)),
    out_shape=jax.ShapeDtypeStruct((NP, F), _f32),
)


def _comb2_body(agg_ref, h_ref, di_ref, w2_ref, b2_ref, o_ref):
    di = di_ref[...]
    a = agg_ref[...] * di + h_ref[...] * (di * di)
    o_ref[...] = (jnp.dot(a, w2_ref[...], preferred_element_type=_f32)
                  + b2_ref[...])


_tc_comb2 = pl.pallas_call(
    _comb2_body,
    grid=(NP // _BR,),
    in_specs=[
        pl.BlockSpec((_BR, F), lambda i: (i, 0)),
        pl.BlockSpec((_BR, F), lambda i: (i, 0)),
        pl.BlockSpec((_BR, 1), lambda i: (i, 0)),
        pl.BlockSpec((F, N_CLASSES), lambda i: (0, 0)),
        pl.BlockSpec((1, N_CLASSES), lambda i: (0, 0)),
    ],
    out_specs=pl.BlockSpec((_BR, N_CLASSES), lambda i: (i, 0)),
    out_shape=jax.ShapeDtypeStruct((NP, N_CLASSES), _f32),
)


def kernel(x, edge_index, edge_weight, W1, b1, W2, b2):
    src = edge_index[0].astype(_i32)
    dst = edge_index[1].astype(_i32)
    pad = EP - N_EDGES
    sd = jnp.bitwise_or(jnp.left_shift(dst, 16), src)
    sd3 = jnp.concatenate([sd, jnp.zeros((pad,), _i32)]).reshape(NW, NCHUNK, K)
    ew3 = jnp.concatenate([edge_weight.astype(_f32),
                           jnp.zeros((pad,), _f32)]).reshape(NW, NCHUNK, K)
    x_p = jnp.zeros((NP, F), _f32).at[:N_NODES].set(x)

    h1 = _tc_matmul(x_p, W1)
    parts1, dinv2d = _sc_layer1(sd3, ew3, h1)
    agg1 = parts1.reshape(NP, F)
    dinv_col = dinv2d.reshape(NP, 1)
    h1r = _tc_comb1(agg1, h1, dinv_col, b1.reshape(1, F))
    parts2 = _sc_layer2(sd3, ew3, h1r, dinv2d)
    agg2 = parts2.reshape(NP, F)
    out_p = _tc_comb2(agg2, h1r, dinv_col, W2, b2.reshape(1, N_CLASSES))
    return out_p[:N_NODES]


# speedup vs baseline: 9.9567x; 1.8216x over previous
import functools

import jax
import jax.numpy as jnp
from jax import lax
from jax.experimental import pallas as pl
from jax.experimental.pallas import tpu as pltpu
from jax.experimental.pallas import tpu_sc as plsc

N_NODES = 10000
N_EDGES = 320000
F = 128
N_CLASSES = 40

NC, NS, NW = 2, 16, 32
NP = 10240
NPR = NP // F
NH = NP // NC
ACCSEG = NH // NS
SEGR = NP // NS // F
K = 128
NCHUNK = 80
EP = NW * NCHUNK * K
CAPW = 11264

_f32 = jnp.float32
_i32 = jnp.int32


def _rsqrt16(x):
    i = plsc.bitcast(x, _i32)
    i = jnp.int32(0x5F3759DF) - lax.shift_right_logical(i, 1)
    y = plsc.bitcast(i, _f32)
    for _ in range(3):
        y = y * (1.5 - 0.5 * x * y * y)
    return y


def _zero_rows(ref, nrows):
    z = jnp.zeros((16,), _f32)

    @pl.loop(0, nrows)
    def _(r):
        for j in range(F // 16):
            ref[r, pl.ds(16 * j, 16)] = z


def _scan_compact(s, c, sd3_h, ew3_h, sdb, ewsb, sdl, ewl, deg_v):
    base = c * NH
    cnt = jnp.int32(0)
    for m in range(2):
        blk = s * 2 + m
        pltpu.sync_copy(sd3_h.at[blk], sdb.at[pl.ds(0, NCHUNK)])
        pltpu.sync_copy(ew3_h.at[blk], ewsb.at[pl.ds(0, NCHUNK)])

        def body(ch, cnt):
            for i in range(K // 16):
                sl = pl.ds(16 * i, 16)
                v = sdb[ch, sl]
                w = ewsb[ch, sl]
                dg = lax.shift_right_logical(v, 16)
                if deg_v is not None:
                    plsc.addupdate_scatter(
                        deg_v,
                        [lax.shift_right_logical(dg, 7),
                         lax.bitwise_and(dg, 127)], w)
                dl = dg - base
                ok = jnp.logical_and(dl >= 0, dl < NH)
                vloc = lax.bitwise_or(lax.shift_left(dl, 16),
                                      lax.bitwise_and(v, 0xFFFF))
                plsc.store_compressed(sdl.at[pl.ds(cnt, 16)], vloc, mask=ok)
                plsc.store_compressed(ewl.at[pl.ds(cnt, 16)], w, mask=ok)
                cnt = jnp.minimum(cnt + jnp.sum(ok.astype(_i32)), CAPW - 128)
            return cnt

        cnt = pl.loop(0, NCHUNK, init_carry=cnt)(body)

    zi = jnp.zeros((16,), _i32)
    zf = jnp.zeros((16,), _f32)
    for j in range(8):
        sdl[pl.ds(cnt + 16 * j, 16)] = zi
        ewl[pl.ds(cnt + 16 * j, 16)] = zf
    return lax.shift_right_logical(cnt + 127, 7)


def _edge_phase(s, c, nch, h_h, parts_h, sdl, ewl, dst2, dinv_v,
                gb0, gb1, accsh, gsem0, gsem1, ssem):
    @pl.loop(0, nch)
    def _(k2):
        for j in range(F // 16):
            sl = pl.ds(k2 * K + 16 * j, 16)
            v = sdl[sl]
            sv = lax.bitwise_and(v, 0xFFFF)
            dst2[k2, pl.ds(16 * j, 16)] = lax.shift_right_logical(v, 16)
            sdl[sl] = sv
            dv = plsc.load_gather(dinv_v, [lax.shift_right_logical(sv, 7),
                                           lax.bitwise_and(sv, 127)])
            ewl[sl] = ewl[sl] * dv

    _zero_rows(gb0, K)
    pltpu.sync_copy(gb0, accsh.at[pl.ds(s * ACCSEG, K)])
    pltpu.sync_copy(gb0, accsh.at[pl.ds(s * ACCSEG + K, K)])
    pltpu.sync_copy(gb0.at[pl.ds(0, ACCSEG - 2 * K)],
                    accsh.at[pl.ds(s * ACCSEG + 2 * K, ACCSEG - 2 * K)])
    plsc.subcore_barrier()

    def _issue(ch, buf, sem):
        pltpu.async_copy(h_h.at[sdl.at[pl.ds(ch * K, K)]], buf, sem)

    def _wait(ch, buf, sem):
        pltpu.make_async_copy(h_h.at[sdl.at[pl.ds(ch * K, K)]], buf,
                              sem).wait()

    def _scale_scatter(ch, buf):
        @pl.loop(0, K)
        def _(e):
            wv = plsc.load_gather(ewl, [jnp.full((16,), ch * K + e, _i32)])
            for j in range(F // 16):
                buf[e, pl.ds(16 * j, 16)] = buf[e, pl.ds(16 * j, 16)] * wv

        pltpu.async_copy(buf, accsh.at[dst2.at[ch]], ssem, add=True).wait()

    @pl.when(nch > 0)
    def _():
        _issue(0, gb0, gsem0)

    @pl.loop(0, lax.shift_right_logical(nch, 1))
    def _(g):
        ch0 = 2 * g
        _issue(ch0 + 1, gb1, gsem1)
        _wait(ch0, gb0, gsem0)
        _scale_scatter(ch0, gb0)

        @pl.when(ch0 + 2 < nch)
        def _():
            _issue(ch0 + 2, gb0, gsem0)

        _wait(ch0 + 1, gb1, gsem1)
        _scale_scatter(ch0 + 1, gb1)

    @pl.when(lax.bitwise_and(nch, 1) == 1)
    def _():
        ch = nch - 1
        _wait(ch, gb0, gsem0)
        _scale_scatter(ch, gb0)

    plsc.subcore_barrier()
    pltpu.sync_copy(accsh.at[pl.ds(s * ACCSEG, ACCSEG)],
                    parts_h.at[c, pl.ds(s * ACCSEG, ACCSEG)])


_SC_PARAMS = pltpu.CompilerParams(needs_layout_passes=False)
_MESH = plsc.VectorSubcoreMesh(core_axis_name="c", subcore_axis_name="s")

_EDGE_SCRATCH = [
    pltpu.VMEM((CAPW // K, K), _i32),
    pltpu.VMEM((CAPW,), _i32),
    pltpu.VMEM((CAPW,), _f32),
    pltpu.VMEM((NPR, F), _f32),
    pltpu.VMEM((K, F), _f32),
    pltpu.VMEM((K, F), _f32),
    pltpu.VMEM_SHARED((NH, F), _f32),
    pltpu.SemaphoreType.DMA,
    pltpu.SemaphoreType.DMA,
    pltpu.SemaphoreType.DMA,
]


@functools.partial(
    pl.kernel,
    out_type=[
        jax.ShapeDtypeStruct((NC, NH, F), _f32),
        jax.ShapeDtypeStruct((NPR, F), _f32),
    ],
    mesh=_MESH,
    compiler_params=_SC_PARAMS,
    scratch_types=_EDGE_SCRATCH + [
        pltpu.VMEM((SEGR, F), _f32),
        pltpu.VMEM((SEGR, F), _f32),
        pltpu.VMEM_SHARED((NPR, F), _f32),
    ],
)
def _sc_layer1(sd3_h, ew3_h, h_h, parts_h, dinv_h,
               sdb, sdl, ewl, dinv_v, gb0, gb1, accsh,
               gsem0, gsem1, ssem, red_v, tmp_v, dinvsh):
    c = lax.axis_index("c")
    s = lax.axis_index("s")

    _zero_rows(dinv_v, NPR)
    nch = _scan_compact(s, c, sd3_h, ew3_h, sdb, gb1, sdl, ewl, dinv_v)

    pltpu.sync_copy(dinv_v, accsh.at[pl.ds(s * NPR, NPR)])
    plsc.subcore_barrier()

    _zero_rows(red_v, SEGR)
    for r in range(NS):
        pltpu.sync_copy(accsh.at[pl.ds(r * NPR + s * SEGR, SEGR)], tmp_v)

        @pl.loop(0, SEGR)
        def _(i):
            for j in range(F // 16):
                red_v[i, pl.ds(16 * j, 16)] = (red_v[i, pl.ds(16 * j, 16)]
                                               + tmp_v[i, pl.ds(16 * j, 16)])

    @pl.loop(0, SEGR)
    def _(i):
        for j in range(F // 16):
            tmp_v[i, pl.ds(16 * j, 16)] = _rsqrt16(
                red_v[i, pl.ds(16 * j, 16)] + 1.0)

    pltpu.sync_copy(tmp_v, dinvsh.at[pl.ds(s * SEGR, SEGR)])
    plsc.subcore_barrier()

    @pl.when(jnp.logical_and(c == 0, s == 0))
    def _():
        pltpu.sync_copy(dinvsh, dinv_h)

    pltpu.sync_copy(dinvsh, dinv_v)

    _edge_phase(s, c, nch, h_h, parts_h, sdl, ewl, sdb, dinv_v,
                gb0, gb1, accsh, gsem0, gsem1, ssem)


@functools.partial(
    pl.kernel,
    out_type=jax.ShapeDtypeStruct((NC, NH, F), _f32),
    mesh=_MESH,
    compiler_params=_SC_PARAMS,
    scratch_types=_EDGE_SCRATCH,
)
def _sc_layer2(sd3_h, ew3_h, h_h, dinv_hin, parts_h,
               sdb, sdl, ewl, dinv_v, gb0, gb1, accsh,
               gsem0, gsem1, ssem):
    c = lax.axis_index("c")
    s = lax.axis_index("s")
    pltpu.sync_copy(dinv_hin, dinv_v)
    nch = _scan_compact(s, c, sd3_h, ew3_h, sdb, gb1, sdl, ewl, None)
    _edge_phase(s, c, nch, h_h, parts_h, sdl, ewl, sdb, dinv_v,
                gb0, gb1, accsh, gsem0, gsem1, ssem)


_BR = 1024


def _mm_body(x_ref, w_ref, o_ref):
    o_ref[...] = jnp.dot(x_ref[...], w_ref[...], preferred_element_type=_f32)


_tc_matmul = pl.pallas_call(
    _mm_body,
    grid=(NP // _BR,),
    in_specs=[
        pl.BlockSpec((_BR, F), lambda i: (i, 0)),
        pl.BlockSpec((F, F), lambda i: (0, 0)),
    ],
    out_specs=pl.BlockSpec((_BR, F), lambda i: (i, 0)),
    out_shape=jax.ShapeDtypeStruct((NP, F), _f32),
)


def _comb1_body(agg_ref, h_ref, di_ref, b_ref, o_ref):
    di = di_ref[...]
    o_ref[...] = jnp.maximum(
        agg_ref[...] * di + h_ref[...] * (di * di) + b_ref[...], 0.0)


_tc_comb1 = pl.pallas_call(
    _comb1_body,
    grid=(NP // _BR,),
    in_specs=[
        pl.BlockSpec((_BR, F), lambda i: (i, 0)),
        pl.BlockSpec((_BR, F), lambda i: (i, 0)),
        pl.BlockSpec((_BR, 1), lambda i: (i, 0)),
        pl.BlockSpec((1, F), lambda i: (0, 0)),
    ],
    out_specs=pl.BlockSpec((_BR, F), lambda i: (i, 0)),
    out_shape=jax.ShapeDtypeStruct((NP, F), _f32),
)


def _comb2_body(agg_ref, h_ref, di_ref, w2_ref, b2_ref, o_ref):
    di = di_ref[...]
    a = agg_ref[...] * di + h_ref[...] * (di * di)
    o_ref[...] = (jnp.dot(a, w2_ref[...], preferred_element_type=_f32)
                  + b2_ref[...])


_tc_comb2 = pl.pallas_call(
    _comb2_body,
    grid=(NP // _BR,),
    in_specs=[
        pl.BlockSpec((_BR, F), lambda i: (i, 0)),
        pl.BlockSpec((_BR, F), lambda i: (i, 0)),
        pl.BlockSpec((_BR, 1), lambda i: (i, 0)),
        pl.BlockSpec((F, N_CLASSES), lambda i: (0, 0)),
        pl.BlockSpec((1, N_CLASSES), lambda i: (0, 0)),
    ],
    out_specs=pl.BlockSpec((_BR, N_CLASSES), lambda i: (i, 0)),
    out_shape=jax.ShapeDtypeStruct((NP, N_CLASSES), _f32),
)


def kernel(x, edge_index, edge_weight, W1, b1, W2, b2):
    src = edge_index[0].astype(_i32)
    dst = edge_index[1].astype(_i32)
    pad = EP - N_EDGES
    sd = jnp.bitwise_or(jnp.left_shift(dst, 16), src)
    sd3 = jnp.concatenate([sd, jnp.zeros((pad,), _i32)]).reshape(NW, NCHUNK, K)
    ew3 = jnp.concatenate([edge_weight.astype(_f32),
                           jnp.zeros((pad,), _f32)]).reshape(NW, NCHUNK, K)
    x_p = jnp.zeros((NP, F), _f32).at[:N_NODES].set(x)

    h1 = _tc_matmul(x_p, W1)
    parts1, dinv2d = _sc_layer1(sd3, ew3, h1)
    agg1 = parts1.reshape(NP, F)
    dinv_col = dinv2d.reshape(NP, 1)
    h1r = _tc_comb1(agg1, h1, dinv_col, b1.reshape(1, F))
    parts2 = _sc_layer2(sd3, ew3, h1r, dinv2d)
    agg2 = parts2.reshape(NP, F)
    out_p = _tc_comb2(agg2, h1r, dinv_col, W2, b2.reshape(1, N_CLASSES))
    return out_p[:N_NODES]


# speedup vs baseline: 10.0744x; 1.0118x over previous
import functools

import jax
import jax.numpy as jnp
from jax import lax
from jax.experimental import pallas as pl
from jax.experimental.pallas import tpu as pltpu
from jax.experimental.pallas import tpu_sc as plsc

N_NODES = 10000
N_EDGES = 320000
F = 128
N_CLASSES = 40

NC, NS, NW = 2, 16, 32
NP = 10240
NPR = NP // F
NH = NP // NC
ACCSEG = NH // NS
SEGR = NP // NS // F
K = 128
NCHUNK = 80
EP = NW * NCHUNK * K
CAPW = 11264

_f32 = jnp.float32
_i32 = jnp.int32


def _rsqrt16(x):
    i = plsc.bitcast(x, _i32)
    i = jnp.int32(0x5F3759DF) - lax.shift_right_logical(i, 1)
    y = plsc.bitcast(i, _f32)
    for _ in range(3):
        y = y * (1.5 - 0.5 * x * y * y)
    return y


def _zero_rows(ref, nrows):
    z = jnp.zeros((16,), _f32)

    @pl.loop(0, nrows)
    def _(r):
        for j in range(F // 16):
            ref[r, pl.ds(16 * j, 16)] = z


def _scan_compact(s, c, sd3_h, ew3_h, sdb, ewsb, sdl, ewl, deg_v):
    base = c * NH
    cnt = jnp.int32(0)
    for m in range(2):
        blk = s * 2 + m
        pltpu.sync_copy(sd3_h.at[blk], sdb.at[pl.ds(0, NCHUNK)])
        pltpu.sync_copy(ew3_h.at[blk], ewsb.at[pl.ds(0, NCHUNK)])

        def body(ch, cnt):
            for i in range(K // 16):
                sl = pl.ds(16 * i, 16)
                v = sdb[ch, sl]
                w = ewsb[ch, sl]
                dg = lax.shift_right_logical(v, 16)
                if deg_v is not None:
                    plsc.addupdate_scatter(
                        deg_v,
                        [lax.shift_right_logical(dg, 7),
                         lax.bitwise_and(dg, 127)], w)
                dl = dg - base
                ok = jnp.logical_and(dl >= 0, dl < NH)
                vloc = lax.bitwise_or(lax.shift_left(dl, 16),
                                      lax.bitwise_and(v, 0xFFFF))
                plsc.store_compressed(sdl.at[pl.ds(cnt, 16)], vloc, mask=ok)
                plsc.store_compressed(ewl.at[pl.ds(cnt, 16)], w, mask=ok)
                cnt = jnp.minimum(cnt + jnp.sum(ok.astype(_i32)), CAPW - 128)
            return cnt

        cnt = pl.loop(0, NCHUNK, init_carry=cnt)(body)

    zi = jnp.zeros((16,), _i32)
    zf = jnp.zeros((16,), _f32)
    for j in range(8):
        sdl[pl.ds(cnt + 16 * j, 16)] = zi
        ewl[pl.ds(cnt + 16 * j, 16)] = zf
    return lax.shift_right_logical(cnt + 127, 7)


def _edge_phase(s, c, nch, h_h, parts_h, sdl, ewl, dst2, dinv_v,
                gb0, gb1, accsh, gsem0, gsem1, ssem):
    @pl.loop(0, nch)
    def _(k2):
        for j in range(F // 16):
            sl = pl.ds(k2 * K + 16 * j, 16)
            v = sdl[sl]
            sv = lax.bitwise_and(v, 0xFFFF)
            dst2[k2, pl.ds(16 * j, 16)] = lax.shift_right_logical(v, 16)
            sdl[sl] = sv
            dv = plsc.load_gather(dinv_v, [lax.shift_right_logical(sv, 7),
                                           lax.bitwise_and(sv, 127)])
            ewl[sl] = ewl[sl] * dv

    _zero_rows(gb0, K)
    pltpu.sync_copy(gb0, accsh.at[pl.ds(s * ACCSEG, K)])
    pltpu.sync_copy(gb0, accsh.at[pl.ds(s * ACCSEG + K, K)])
    pltpu.sync_copy(gb0.at[pl.ds(0, ACCSEG - 2 * K)],
                    accsh.at[pl.ds(s * ACCSEG + 2 * K, ACCSEG - 2 * K)])
    plsc.subcore_barrier()

    def _issue(ch, buf, sem):
        pltpu.async_copy(h_h.at[sdl.at[pl.ds(ch * K, K)]], buf, sem)

    def _wait(ch, buf, sem):
        pltpu.make_async_copy(h_h.at[sdl.at[pl.ds(ch * K, K)]], buf,
                              sem).wait()

    def _scale_scatter(ch, buf):
        @pl.loop(0, K, unroll=2)
        def _(e):
            wv = plsc.load_gather(ewl, [jnp.full((16,), ch * K + e, _i32)])
            for j in range(F // 16):
                buf[e, pl.ds(16 * j, 16)] = buf[e, pl.ds(16 * j, 16)] * wv

        pltpu.async_copy(buf, accsh.at[dst2.at[ch]], ssem, add=True).wait()

    @pl.when(nch > 0)
    def _():
        _issue(0, gb0, gsem0)

    @pl.loop(0, lax.shift_right_logical(nch, 1))
    def _(g):
        ch0 = 2 * g
        _issue(ch0 + 1, gb1, gsem1)
        _wait(ch0, gb0, gsem0)
        _scale_scatter(ch0, gb0)

        @pl.when(ch0 + 2 < nch)
        def _():
            _issue(ch0 + 2, gb0, gsem0)

        _wait(ch0 + 1, gb1, gsem1)
        _scale_scatter(ch0 + 1, gb1)

    @pl.when(lax.bitwise_and(nch, 1) == 1)
    def _():
        ch = nch - 1
        _wait(ch, gb0, gsem0)
        _scale_scatter(ch, gb0)

    plsc.subcore_barrier()
    pltpu.sync_copy(accsh.at[pl.ds(s * ACCSEG, ACCSEG)],
                    parts_h.at[c, pl.ds(s * ACCSEG, ACCSEG)])


_SC_PARAMS = pltpu.CompilerParams(needs_layout_passes=False)
_MESH = plsc.VectorSubcoreMesh(core_axis_name="c", subcore_axis_name="s")

_EDGE_SCRATCH = [
    pltpu.VMEM((CAPW // K, K), _i32),
    pltpu.VMEM((CAPW,), _i32),
    pltpu.VMEM((CAPW,), _f32),
    pltpu.VMEM((NPR, F), _f32),
    pltpu.VMEM((K, F), _f32),
    pltpu.VMEM((K, F), _f32),
    pltpu.VMEM_SHARED((NH, F), _f32),
    pltpu.SemaphoreType.DMA,
    pltpu.SemaphoreType.DMA,
    pltpu.SemaphoreType.DMA,
]


@functools.partial(
    pl.kernel,
    out_type=[
        jax.ShapeDtypeStruct((NC, NH, F), _f32),
        jax.ShapeDtypeStruct((NPR, F), _f32),
    ],
    mesh=_MESH,
    compiler_params=_SC_PARAMS,
    scratch_types=_EDGE_SCRATCH + [
        pltpu.VMEM((SEGR, F), _f32),
        pltpu.VMEM((SEGR, F), _f32),
        pltpu.VMEM_SHARED((NPR, F), _f32),
    ],
)
def _sc_layer1(sd3_h, ew3_h, h_h, parts_h, dinv_h,
               sdb, sdl, ewl, dinv_v, gb0, gb1, accsh,
               gsem0, gsem1, ssem, red_v, tmp_v, dinvsh):
    c = lax.axis_index("c")
    s = lax.axis_index("s")

    _zero_rows(dinv_v, NPR)
    nch = _scan_compact(s, c, sd3_h, ew3_h, sdb, gb1, sdl, ewl, dinv_v)

    pltpu.sync_copy(dinv_v, accsh.at[pl.ds(s * NPR, NPR)])
    plsc.subcore_barrier()

    _zero_rows(red_v, SEGR)
    for r in range(NS):
        pltpu.sync_copy(accsh.at[pl.ds(r * NPR + s * SEGR, SEGR)], tmp_v)

        @pl.loop(0, SEGR)
        def _(i):
            for j in range(F // 16):
                red_v[i, pl.ds(16 * j, 16)] = (red_v[i, pl.ds(16 * j, 16)]
                                               + tmp_v[i, pl.ds(16 * j, 16)])

    @pl.loop(0, SEGR)
    def _(i):
        for j in range(F // 16):
            tmp_v[i, pl.ds(16 * j, 16)] = _rsqrt16(
                red_v[i, pl.ds(16 * j, 16)] + 1.0)

    pltpu.sync_copy(tmp_v, dinvsh.at[pl.ds(s * SEGR, SEGR)])
    plsc.subcore_barrier()

    @pl.when(jnp.logical_and(c == 0, s == 0))
    def _():
        pltpu.sync_copy(dinvsh, dinv_h)

    pltpu.sync_copy(dinvsh, dinv_v)

    _edge_phase(s, c, nch, h_h, parts_h, sdl, ewl, sdb, dinv_v,
                gb0, gb1, accsh, gsem0, gsem1, ssem)


@functools.partial(
    pl.kernel,
    out_type=jax.ShapeDtypeStruct((NC, NH, F), _f32),
    mesh=_MESH,
    compiler_params=_SC_PARAMS,
    scratch_types=_EDGE_SCRATCH,
)
def _sc_layer2(sd3_h, ew3_h, h_h, dinv_hin, parts_h,
               sdb, sdl, ewl, dinv_v, gb0, gb1, accsh,
               gsem0, gsem1, ssem):
    c = lax.axis_index("c")
    s = lax.axis_index("s")
    pltpu.sync_copy(dinv_hin, dinv_v)
    nch = _scan_compact(s, c, sd3_h, ew3_h, sdb, gb1, sdl, ewl, None)
    _edge_phase(s, c, nch, h_h, parts_h, sdl, ewl, sdb, dinv_v,
                gb0, gb1, accsh, gsem0, gsem1, ssem)


_BR = 1024


def _mm_body(x_ref, w_ref, o_ref):
    o_ref[...] = jnp.dot(x_ref[...], w_ref[...], preferred_element_type=_f32)


_tc_matmul = pl.pallas_call(
    _mm_body,
    grid=(NP // _BR,),
    in_specs=[
        pl.BlockSpec((_BR, F), lambda i: (i, 0)),
        pl.BlockSpec((F, F), lambda i: (0, 0)),
    ],
    out_specs=pl.BlockSpec((_BR, F), lambda i: (i, 0)),
    out_shape=jax.ShapeDtypeStruct((NP, F), _f32),
)


def _comb1_body(agg_ref, h_ref, di_ref, b_ref, o_ref):
    di = di_ref[...]
    o_ref[...] = jnp.maximum(
        agg_ref[...] * di + h_ref[...] * (di * di) + b_ref[...], 0.0)


_tc_comb1 = pl.pallas_call(
    _comb1_body,
    grid=(NP // _BR,),
    in_specs=[
        pl.BlockSpec((_BR, F), lambda i: (i, 0)),
        pl.BlockSpec((_BR, F), lambda i: (i, 0)),
        pl.BlockSpec((_BR, 1), lambda i: (i, 0)),
        pl.BlockSpec((1, F), lambda i: (0, 0)),
    ],
    out_specs=pl.BlockSpec((_BR, F), lambda i: (i, 0)),
    out_shape=jax.ShapeDtypeStruct((NP, F), _f32),
)


def _comb2_body(agg_ref, h_ref, di_ref, w2_ref, b2_ref, o_ref):
    di = di_ref[...]
    a = agg_ref[...] * di + h_ref[...] * (di * di)
    o_ref[...] = (jnp.dot(a, w2_ref[...], preferred_element_type=_f32)
                  + b2_ref[...])


_tc_comb2 = pl.pallas_call(
    _comb2_body,
    grid=(NP // _BR,),
    in_specs=[
        pl.BlockSpec((_BR, F), lambda i: (i, 0)),
        pl.BlockSpec((_BR, F), lambda i: (i, 0)),
        pl.BlockSpec((_BR, 1), lambda i: (i, 0)),
        pl.BlockSpec((F, N_CLASSES), lambda i: (0, 0)),
        pl.BlockSpec((1, N_CLASSES), lambda i: (0, 0)),
    ],
    out_specs=pl.BlockSpec((_BR, N_CLASSES), lambda i: (i, 0)),
    out_shape=jax.ShapeDtypeStruct((NP, N_CLASSES), _f32),
)


def kernel(x, edge_index, edge_weight, W1, b1, W2, b2):
    src = edge_index[0].astype(_i32)
    dst = edge_index[1].astype(_i32)
    pad = EP - N_EDGES
    sd = jnp.bitwise_or(jnp.left_shift(dst, 16), src)
    sd3 = jnp.concatenate([sd, jnp.zeros((pad,), _i32)]).reshape(NW, NCHUNK, K)
    ew3 = jnp.concatenate([edge_weight.astype(_f32),
                           jnp.zeros((pad,), _f32)]).reshape(NW, NCHUNK, K)
    x_p = jnp.zeros((NP, F), _f32).at[:N_NODES].set(x)

    h1 = _tc_matmul(x_p, W1)
    parts1, dinv2d = _sc_layer1(sd3, ew3, h1)
    agg1 = parts1.reshape(NP, F)
    dinv_col = dinv2d.reshape(NP, 1)
    h1r = _tc_comb1(agg1, h1, dinv_col, b1.reshape(1, F))
    parts2 = _sc_layer2(sd3, ew3, h1r, dinv2d)
    agg2 = parts2.reshape(NP, F)
    out_p = _tc_comb2(agg2, h1r, dinv_col, W2, b2.reshape(1, N_CLASSES))
    return out_p[:N_NODES]
